# Initial kernel scaffold; baseline (speedup 1.0000x reference)
import jax
import jax.numpy as jnp
from jax.experimental import pallas as pl


def kernel(input_hidden, hidden, last_hidden, batch_assignment, edge_index, W1, b1, W2, b2):
    raise NotImplementedError("write your pallas kernel here")


# speedup vs baseline: 6.5907x; 6.5907x over previous
import functools

import jax
import jax.numpy as jnp
from jax import lax
from jax.experimental import pallas as pl
from jax.experimental.pallas import tpu as pltpu
from jax.experimental.pallas import tpu_sc as plsc

N = 10000
E = 160000
H = 256
IN = 2 * H
HH = H // 2

NC = 2
NS = 16
K = 128
NB = 79
EPT = NB * K
E_PAD = EPT * NS
DUMMY = N
N_ACC = 10240

ROW_BLK = 1000
GRID_I = N // ROW_BLK


def _stage1_body(ih_ref, h_ref, w1_ref, out_ref):
    w = w1_ref[...]
    out_ref[...] = (
        jnp.dot(ih_ref[...], w[:H, :], preferred_element_type=jnp.float32)
        + jnp.dot(h_ref[...], w[H:, :], preferred_element_type=jnp.float32)
    )


def _stage1(input_hidden, hidden, W1):
    return pl.pallas_call(
        _stage1_body,
        grid=(GRID_I, NC),
        in_specs=[
            pl.BlockSpec((ROW_BLK, H), lambda i, j: (i, 0)),
            pl.BlockSpec((ROW_BLK, H), lambda i, j: (i, 0)),
            pl.BlockSpec((IN, HH), lambda i, j: (0, j)),
        ],
        out_specs=pl.BlockSpec((ROW_BLK, HH), lambda i, j: (j * GRID_I + i, 0)),
        out_shape=jax.ShapeDtypeStruct((NC * N, HH), jnp.float32),
    )(input_hidden, hidden, W1)


def _sc_body(y_hbm, src_hbm, dst_hbm, out_hbm,
             srcbuf, dstbuf, srcoff, rows, acc, sem):
    c = lax.axis_index("c")
    s = lax.axis_index("s")

    pltpu.sync_copy(src_hbm.at[s], srcbuf)
    pltpu.sync_copy(dst_hbm.at[s], dstbuf)

    off = c * N

    def _off_body(j, carry):
        for i in range(K // 16):
            sl = pl.ds(i * 16, 16)
            srcoff[j, sl] = srcbuf[j, sl] + off
        return carry

    lax.fori_loop(0, NB, _off_body, 0)

    zvec = jnp.zeros((16,), jnp.float32)

    def _z_body(j, carry):
        for i in range(HH // 16):
            rows[j, pl.ds(i * 16, 16)] = zvec
        return carry

    lax.fori_loop(0, K, _z_body, 0)
    for kk in range(N_ACC // NS // K):
        pltpu.sync_copy(rows, acc.at[pl.ds(s * (N_ACC // NS) + kk * K, K)])
    plsc.subcore_barrier()

    def _main_body(j, carry):
        pltpu.async_copy(y_hbm.at[srcoff.at[j]], rows, sem).wait()
        pltpu.sync_copy(rows, acc.at[dstbuf.at[j]], add=True)
        return carry

    lax.fori_loop(0, NB, _main_body, 0)
    plsc.subcore_barrier()

    @pl.when(s < NS - 1)
    def _():
        pltpu.sync_copy(acc.at[pl.ds(s * 632, 632)],
                        out_hbm.at[pl.ds(c * N + s * 632, 632)])

    @pl.when(s == NS - 1)
    def _():
        pltpu.sync_copy(acc.at[pl.ds((NS - 1) * 632, 520)],
                        out_hbm.at[pl.ds(c * N + (NS - 1) * 632, 520)])


def _stage2(y_split, src_r, dst_r):
    mesh = plsc.VectorSubcoreMesh(core_axis_name="c", subcore_axis_name="s")
    f = functools.partial(
        pl.kernel,
        mesh=mesh,
        out_type=jax.ShapeDtypeStruct((NC * N, HH), jnp.float32),
        scratch_types=[
            pltpu.VMEM((NB, K), jnp.int32),
            pltpu.VMEM((NB, K), jnp.int32),
            pltpu.VMEM((NB, K), jnp.int32),
            pltpu.VMEM((K, HH), jnp.float32),
            pltpu.VMEM_SHARED((N_ACC, HH), jnp.float32),
            pltpu.SemaphoreType.DMA,
        ],
    )(_sc_body)
    return f(y_split, src_r, dst_r)


def _stage3_body(y0_ref, y1_ref, a0_ref, a1_ref, b1_ref, w2_ref, b2_ref, out_ref):
    z = jnp.concatenate(
        [y0_ref[...] + a0_ref[...], y1_ref[...] + a1_ref[...]], axis=1)
    hmat = jnp.maximum(z + b1_ref[...], 0.0)
    out_ref[...] = (
        jnp.dot(hmat, w2_ref[...], preferred_element_type=jnp.float32)
        + b2_ref[...]
    )


def _stage3(y_split, agg_split, b1, W2, b2):
    return pl.pallas_call(
        _stage3_body,
        grid=(GRID_I,),
        in_specs=[
            pl.BlockSpec((ROW_BLK, HH), lambda i: (i, 0)),
            pl.BlockSpec((ROW_BLK, HH), lambda i: (GRID_I + i, 0)),
            pl.BlockSpec((ROW_BLK, HH), lambda i: (i, 0)),
            pl.BlockSpec((ROW_BLK, HH), lambda i: (GRID_I + i, 0)),
            pl.BlockSpec((1, H), lambda i: (0, 0)),
            pl.BlockSpec((H, H), lambda i: (0, 0)),
            pl.BlockSpec((1, H), lambda i: (0, 0)),
        ],
        out_specs=pl.BlockSpec((ROW_BLK, H), lambda i: (i, 0)),
        out_shape=jax.ShapeDtypeStruct((N, H), jnp.float32),
    )(y_split, y_split, agg_split, agg_split, b1, W2, b2)


def kernel(input_hidden, hidden, last_hidden, batch_assignment, edge_index,
           W1, b1, W2, b2):
    y_split = _stage1(input_hidden, hidden, W1)

    pad = E_PAD - E
    src_p = jnp.concatenate([edge_index[0], jnp.zeros((pad,), jnp.int32)])
    dst_p = jnp.concatenate([edge_index[1], jnp.full((pad,), DUMMY, jnp.int32)])
    src_r = src_p.reshape(NS, NB, K)
    dst_r = dst_p.reshape(NS, NB, K)

    agg_split = _stage2(y_split, src_r, dst_r)

    return _stage3(y_split, agg_split,
                   b1.reshape(1, H), W2, b2.reshape(1, H))
